# baseline (speedup 1.0000x reference)
import functools

import jax
import jax.numpy as jnp
from jax.experimental import pallas as pl
from jax.experimental.pallas import tpu as pltpu

SCALE = 0.125
BIG = 1024.0
MINIT = -1e30


def _attn_kernel(cu_q_ref, cu_k_ref, q_ref, k_ref, v_ref, o_ref, *, bq, bk, nbounds):
    i = pl.program_id(1)
    row0 = i * bq
    qb = q_ref[0]

    seg_first = 0
    seg_last = 0
    for b in range(1, nbounds):
        bound = cu_q_ref[b]
        seg_first += jnp.where(row0 >= bound, 1, 0)
        seg_last += jnp.where(row0 + bq - 1 >= bound, 1, 0)
    lo = cu_k_ref[seg_first]
    hi = cu_k_ref[seg_last + 1]
    jlo = lo // bk
    jhi = (hi + bk - 1) // bk

    def body(j, carry):
        acc, m, l = carry
        col0 = j * bk
        kb = k_ref[0, pl.ds(col0, bk), :]
        s = jax.lax.dot_general(qb, kb, (((1,), (1,)), ((), ())),
                                preferred_element_type=jnp.float32)
        m_new = jnp.maximum(m, jnp.max(s, axis=1, keepdims=True))
        p = jnp.exp(s - m_new)
        alpha = jnp.exp(m - m_new)
        l_new = l * alpha + jnp.sum(p, axis=1, keepdims=True)
        vb = v_ref[0, pl.ds(col0, bk), :]
        acc_new = acc * alpha + jax.lax.dot_general(
            p, vb, (((1,), (0,)), ((), ())), preferred_element_type=jnp.float32)
        return acc_new, m_new, l_new

    d = v_ref.shape[2]
    acc0 = jnp.zeros((bq, d), jnp.float32)
    m0 = jnp.full((bq, 1), MINIT, jnp.float32)
    l0 = jnp.zeros((bq, 1), jnp.float32)
    acc, _, l = jax.lax.fori_loop(jlo, jhi, body, (acc0, m0, l0))
    o_ref[0] = acc / l


def kernel(q, k, v, cu_seqlens_q, cu_seqlens_k):
    t, h, d = q.shape
    hk = k.shape[1]
    rep = h // hk
    bq = 256
    bk = 256
    nbounds = cu_seqlens_q.shape[0]
    nseg = nbounds - 1
    daug = 80

    cu_q = cu_seqlens_q.astype(jnp.int32)
    cu_k = cu_seqlens_k.astype(jnp.int32)

    rows = jnp.arange(t, dtype=jnp.int32)
    seg_q = jnp.searchsorted(cu_q[1:], rows, side='right').astype(jnp.int32)
    seg_k = jnp.searchsorted(cu_k[1:], rows, side='right').astype(jnp.int32)
    sid = jnp.arange(nseg, dtype=jnp.int32)
    oh_q = (seg_q[:, None] == sid[None, :]).astype(jnp.float32)
    oh_k = (seg_k[:, None] == sid[None, :]).astype(jnp.float32)

    qh = jnp.transpose(q, (1, 0, 2)) * SCALE
    kh = jnp.transpose(k, (1, 0, 2))
    vh = jnp.transpose(v, (1, 0, 2))
    pad_q = jnp.zeros((t, daug - d - nseg - 1), jnp.float32)
    q_ext = jnp.concatenate(
        [BIG * oh_q, jnp.full((t, 1), -BIG, jnp.float32), pad_q], axis=1)
    k_ext = jnp.concatenate(
        [oh_k, jnp.ones((t, 1), jnp.float32), pad_q], axis=1)
    qa = jnp.concatenate([qh, jnp.broadcast_to(q_ext, (h, t, daug - d))], axis=2)
    ka = jnp.concatenate([kh, jnp.broadcast_to(k_ext, (hk, t, daug - d))], axis=2)

    grid = (h, t // bq)
    out = pl.pallas_call(
        functools.partial(_attn_kernel, bq=bq, bk=bk, nbounds=nbounds),
        grid_spec=pltpu.PrefetchScalarGridSpec(
            num_scalar_prefetch=2,
            grid=grid,
            in_specs=[
                pl.BlockSpec((1, bq, daug), lambda hh, ii, *_: (hh, ii, 0)),
                pl.BlockSpec((1, t, daug), lambda hh, ii, *_: (hh // rep, 0, 0)),
                pl.BlockSpec((1, t, d), lambda hh, ii, *_: (hh // rep, 0, 0)),
            ],
            out_specs=pl.BlockSpec((1, bq, d), lambda hh, ii, *_: (hh, ii, 0)),
        ),
        out_shape=jax.ShapeDtypeStruct((h, t, d), jnp.float32),
    )(cu_q, cu_k, qa, ka, vh)
    return jnp.transpose(out, (1, 0, 2)).astype(q.dtype)


# speedup vs baseline: 1.2057x; 1.2057x over previous
import functools

import jax
import jax.numpy as jnp
from jax.experimental import pallas as pl
from jax.experimental.pallas import tpu as pltpu

SCALE = 0.125
NEG = -1e30


def _attn_kernel(cu_q_ref, cu_k_ref, q_ref, k_ref, v_ref, o_ref, *, bq, bk, nbounds):
    i = pl.program_id(1)
    row0 = i * bq
    qb = q_ref[0]

    rows = row0 + jax.lax.broadcasted_iota(jnp.int32, (bq, 1), 0)
    seg_q = jnp.zeros((bq, 1), jnp.int32)
    seg_first = 0
    seg_last = 0
    for b in range(1, nbounds):
        bound = cu_q_ref[b]
        seg_q += (rows >= bound).astype(jnp.int32)
        seg_first += jnp.where(row0 >= bound, 1, 0)
        seg_last += jnp.where(row0 + bq - 1 >= bound, 1, 0)

    lo = cu_k_ref[seg_first]
    hi = cu_k_ref[seg_last + 1]
    jlo = lo // bk
    jhi = (hi + bk - 1) // bk

    def body(j, carry):
        acc, m, l = carry
        col0 = j * bk
        kb = k_ref[0, pl.ds(col0, bk), :]
        s = jax.lax.dot_general(qb, kb, (((1,), (1,)), ((), ())),
                                preferred_element_type=jnp.float32)
        cols = col0 + jax.lax.broadcasted_iota(jnp.int32, (1, bk), 1)
        seg_k = jnp.zeros((1, bk), jnp.int32)
        for b in range(1, nbounds):
            seg_k += (cols >= cu_k_ref[b]).astype(jnp.int32)
        s = jnp.where(seg_q == seg_k, s, NEG)
        m_new = jnp.maximum(m, jnp.max(s, axis=1, keepdims=True))
        p = jnp.exp(s - m_new)
        alpha = jnp.exp(m - m_new)
        l_new = l * alpha + jnp.sum(p, axis=1, keepdims=True)
        vb = v_ref[0, pl.ds(col0, bk), :]
        acc_new = acc * alpha + jax.lax.dot_general(
            p, vb, (((1,), (0,)), ((), ())), preferred_element_type=jnp.float32)
        return acc_new, m_new, l_new

    d = q_ref.shape[2]
    acc0 = jnp.zeros((bq, d), jnp.float32)
    m0 = jnp.full((bq, 1), NEG, jnp.float32)
    l0 = jnp.zeros((bq, 1), jnp.float32)
    acc, _, l = jax.lax.fori_loop(jlo, jhi, body, (acc0, m0, l0))
    o_ref[0] = acc / l


def kernel(q, k, v, cu_seqlens_q, cu_seqlens_k):
    t, h, d = q.shape
    hk = k.shape[1]
    rep = h // hk
    bq = 256
    bk = 256
    nbounds = cu_seqlens_q.shape[0]

    qh = jnp.transpose(q, (1, 0, 2)) * SCALE
    kh = jnp.transpose(k, (1, 0, 2))
    vh = jnp.transpose(v, (1, 0, 2))

    grid = (h, t // bq)
    out = pl.pallas_call(
        functools.partial(_attn_kernel, bq=bq, bk=bk, nbounds=nbounds),
        grid_spec=pltpu.PrefetchScalarGridSpec(
            num_scalar_prefetch=2,
            grid=grid,
            in_specs=[
                pl.BlockSpec((1, bq, d), lambda hh, ii, *_: (hh, ii, 0)),
                pl.BlockSpec((1, t, d), lambda hh, ii, *_: (hh // rep, 0, 0)),
                pl.BlockSpec((1, t, d), lambda hh, ii, *_: (hh // rep, 0, 0)),
            ],
            out_specs=pl.BlockSpec((1, bq, d), lambda hh, ii, *_: (hh, ii, 0)),
        ),
        out_shape=jax.ShapeDtypeStruct((h, t, d), jnp.float32),
    )(cu_seqlens_q.astype(jnp.int32), cu_seqlens_k.astype(jnp.int32), qh, kh, vh)
    return jnp.transpose(out, (1, 0, 2)).astype(q.dtype)


# speedup vs baseline: 1.4781x; 1.2260x over previous
import functools

import jax
import jax.numpy as jnp
from jax.experimental import pallas as pl
from jax.experimental.pallas import tpu as pltpu

SCALE = 0.125
NEG = -1e30


def _attn_kernel(cu_q_ref, cu_k_ref, q_ref, k_ref, v_ref, o_ref, *, bq, bk, nbounds):
    i = pl.program_id(1)
    row0 = i * bq
    qb = q_ref[0]

    rows = row0 + jax.lax.broadcasted_iota(jnp.int32, (bq, 1), 0)
    seg_q = jnp.zeros((bq, 1), jnp.int32)
    seg_first = 0
    seg_last = 0
    for b in range(1, nbounds):
        bound = cu_q_ref[b]
        seg_q += (rows >= bound).astype(jnp.int32)
        seg_first += jnp.where(row0 >= bound, 1, 0)
        seg_last += jnp.where(row0 + bq - 1 >= bound, 1, 0)

    lo = cu_k_ref[seg_first]
    hi = cu_k_ref[seg_last + 1]
    jlo = lo // bk
    jhi = (hi + bk - 1) // bk

    def body(j, carry):
        acc, m, l = carry
        col0 = j * bk
        kb = k_ref[0, pl.ds(col0, bk), :]
        s = jax.lax.dot_general(qb, kb, (((1,), (1,)), ((), ())),
                                preferred_element_type=jnp.float32)
        cols = col0 + jax.lax.broadcasted_iota(jnp.int32, (1, bk), 1)
        seg_k = jnp.zeros((1, bk), jnp.int32)
        for b in range(1, nbounds):
            seg_k += (cols >= cu_k_ref[b]).astype(jnp.int32)
        s = jnp.where(seg_q == seg_k, s, NEG)
        m_new = jnp.maximum(m, jnp.max(s, axis=1, keepdims=True))
        p = jnp.exp(s - m_new)
        alpha = jnp.exp(m - m_new)
        l_new = l * alpha + jnp.sum(p, axis=1, keepdims=True)
        vb = v_ref[0, pl.ds(col0, bk), :]
        acc_new = acc * alpha + jax.lax.dot_general(
            p, vb, (((1,), (0,)), ((), ())), preferred_element_type=jnp.float32)
        return acc_new, m_new, l_new

    d = q_ref.shape[2]
    acc0 = jnp.zeros((bq, d), jnp.float32)
    m0 = jnp.full((bq, 1), NEG, jnp.float32)
    l0 = jnp.zeros((bq, 1), jnp.float32)
    acc, _, l = jax.lax.fori_loop(jlo, jhi, body, (acc0, m0, l0))
    o_ref[0] = acc / l


def kernel(q, k, v, cu_seqlens_q, cu_seqlens_k):
    t, h, d = q.shape
    hk = k.shape[1]
    rep = h // hk
    bq = 256
    bk = 512
    nbounds = cu_seqlens_q.shape[0]

    qh = jnp.transpose(q, (1, 0, 2)) * SCALE
    kh = jnp.transpose(k, (1, 0, 2))
    vh = jnp.transpose(v, (1, 0, 2))

    grid = (h, t // bq)
    out = pl.pallas_call(
        functools.partial(_attn_kernel, bq=bq, bk=bk, nbounds=nbounds),
        grid_spec=pltpu.PrefetchScalarGridSpec(
            num_scalar_prefetch=2,
            grid=grid,
            in_specs=[
                pl.BlockSpec((1, bq, d), lambda hh, ii, *_: (hh, ii, 0)),
                pl.BlockSpec((1, t, d), lambda hh, ii, *_: (hh // rep, 0, 0)),
                pl.BlockSpec((1, t, d), lambda hh, ii, *_: (hh // rep, 0, 0)),
            ],
            out_specs=pl.BlockSpec((1, bq, d), lambda hh, ii, *_: (hh, ii, 0)),
        ),
        out_shape=jax.ShapeDtypeStruct((h, t, d), jnp.float32),
    )(cu_seqlens_q.astype(jnp.int32), cu_seqlens_k.astype(jnp.int32), qh, kh, vh)
    return jnp.transpose(out, (1, 0, 2)).astype(q.dtype)


# speedup vs baseline: 1.5142x; 1.0244x over previous
import functools

import jax
import jax.numpy as jnp
from jax.experimental import pallas as pl
from jax.experimental.pallas import tpu as pltpu

SCALE = 0.125
NEG = -1e30


def _attn_kernel(cu_q_ref, cu_k_ref, q_ref, k_ref, v_ref, o_ref, *, bq, bk, nbounds):
    i = pl.program_id(1)
    row0 = i * bq
    qb = q_ref[0]

    rows = row0 + jax.lax.broadcasted_iota(jnp.int32, (bq, 1), 0)
    seg_q = jnp.zeros((bq, 1), jnp.int32)
    seg_first = 0
    seg_last = 0
    for b in range(1, nbounds):
        bound = cu_q_ref[b]
        seg_q += (rows >= bound).astype(jnp.int32)
        seg_first += jnp.where(row0 >= bound, 1, 0)
        seg_last += jnp.where(row0 + bq - 1 >= bound, 1, 0)

    lo = cu_k_ref[seg_first]
    hi = cu_k_ref[seg_last + 1]
    jlo = lo // bk
    jhi = (hi + bk - 1) // bk

    def body(j, carry):
        acc, m, l = carry
        col0 = j * bk
        kb = k_ref[0, pl.ds(col0, bk), :]
        s = jax.lax.dot_general(qb, kb, (((1,), (1,)), ((), ())),
                                preferred_element_type=jnp.float32)
        cols = col0 + jax.lax.broadcasted_iota(jnp.int32, (1, bk), 1)
        seg_k = jnp.zeros((1, bk), jnp.int32)
        for b in range(1, nbounds):
            seg_k += (cols >= cu_k_ref[b]).astype(jnp.int32)
        s = jnp.where(seg_q == seg_k, s, NEG)
        m_new = jnp.maximum(m, jnp.max(s, axis=1, keepdims=True))
        p = jnp.exp(s - m_new)
        alpha = jnp.exp(m - m_new)
        l_new = l * alpha + jnp.sum(p, axis=1, keepdims=True)
        vb = v_ref[0, pl.ds(col0, bk), :]
        acc_new = acc * alpha + jax.lax.dot_general(
            p, vb, (((1,), (0,)), ((), ())), preferred_element_type=jnp.float32)
        return acc_new, m_new, l_new

    d = q_ref.shape[2]
    acc0 = jnp.zeros((bq, d), jnp.float32)
    m0 = jnp.full((bq, 1), NEG, jnp.float32)
    l0 = jnp.zeros((bq, 1), jnp.float32)
    acc, _, l = jax.lax.fori_loop(jlo, jhi, body, (acc0, m0, l0))
    o_ref[0] = acc / l


def kernel(q, k, v, cu_seqlens_q, cu_seqlens_k):
    t, h, d = q.shape
    hk = k.shape[1]
    rep = h // hk
    bq = 256
    bk = 1024
    nbounds = cu_seqlens_q.shape[0]

    qh = jnp.transpose(q, (1, 0, 2)) * SCALE
    kh = jnp.transpose(k, (1, 0, 2))
    vh = jnp.transpose(v, (1, 0, 2))

    grid = (h, t // bq)
    out = pl.pallas_call(
        functools.partial(_attn_kernel, bq=bq, bk=bk, nbounds=nbounds),
        grid_spec=pltpu.PrefetchScalarGridSpec(
            num_scalar_prefetch=2,
            grid=grid,
            in_specs=[
                pl.BlockSpec((1, bq, d), lambda hh, ii, *_: (hh, ii, 0)),
                pl.BlockSpec((1, t, d), lambda hh, ii, *_: (hh // rep, 0, 0)),
                pl.BlockSpec((1, t, d), lambda hh, ii, *_: (hh // rep, 0, 0)),
            ],
            out_specs=pl.BlockSpec((1, bq, d), lambda hh, ii, *_: (hh, ii, 0)),
        ),
        out_shape=jax.ShapeDtypeStruct((h, t, d), jnp.float32),
    )(cu_seqlens_q.astype(jnp.int32), cu_seqlens_k.astype(jnp.int32), qh, kh, vh)
    return jnp.transpose(out, (1, 0, 2)).astype(q.dtype)


# speedup vs baseline: 1.6410x; 1.0837x over previous
import functools

import jax
import jax.numpy as jnp
from jax.experimental import pallas as pl
from jax.experimental.pallas import tpu as pltpu

SCALE = 0.125
NEG = -1e30


def _attn_kernel(cu_q_ref, cu_k_ref, q_ref, k_ref, v_ref, o_ref, *, bq, bk, nbounds):
    i = pl.program_id(1)
    row0 = i * bq
    qb = q_ref[0]

    rows = row0 + jax.lax.broadcasted_iota(jnp.int32, (bq, 1), 0)
    seg_q = jnp.zeros((bq, 1), jnp.int32)
    seg_first = 0
    seg_last = 0
    for b in range(1, nbounds):
        bound = cu_q_ref[b]
        seg_q += (rows >= bound).astype(jnp.int32)
        seg_first += jnp.where(row0 >= bound, 1, 0)
        seg_last += jnp.where(row0 + bq - 1 >= bound, 1, 0)

    lo = cu_k_ref[seg_first]
    hi = cu_k_ref[seg_last + 1]
    jlo = lo // bk
    jhi = (hi + bk - 1) // bk

    def body(j, carry):
        acc, m, l = carry
        col0 = j * bk
        kb = k_ref[0, pl.ds(col0, bk), :]
        s = jax.lax.dot_general(qb, kb, (((1,), (1,)), ((), ())),
                                preferred_element_type=jnp.float32)
        cols = col0 + jax.lax.broadcasted_iota(jnp.int32, (1, bk), 1)
        seg_k = jnp.zeros((1, bk), jnp.int32)
        for b in range(1, nbounds):
            seg_k += (cols >= cu_k_ref[b]).astype(jnp.int32)
        s = jnp.where(seg_q == seg_k, s, NEG)
        m_new = jnp.maximum(m, jnp.max(s, axis=1, keepdims=True))
        p = jnp.exp(s - m_new)
        alpha = jnp.exp(m - m_new)
        l_new = l * alpha + jnp.sum(p, axis=1, keepdims=True)
        vb = v_ref[0, pl.ds(col0, bk), :]
        acc_new = acc * alpha + jax.lax.dot_general(
            p, vb, (((1,), (0,)), ((), ())), preferred_element_type=jnp.float32)
        return acc_new, m_new, l_new

    d = q_ref.shape[2]
    acc0 = jnp.zeros((bq, d), jnp.float32)
    m0 = jnp.full((bq, 1), NEG, jnp.float32)
    l0 = jnp.zeros((bq, 1), jnp.float32)
    acc, _, l = jax.lax.fori_loop(jlo, jhi, body, (acc0, m0, l0))
    o_ref[0] = acc / l


def kernel(q, k, v, cu_seqlens_q, cu_seqlens_k):
    t, h, d = q.shape
    hk = k.shape[1]
    rep = h // hk
    bq = 512
    bk = 1024
    nbounds = cu_seqlens_q.shape[0]

    qh = jnp.transpose(q, (1, 0, 2)) * SCALE
    kh = jnp.transpose(k, (1, 0, 2))
    vh = jnp.transpose(v, (1, 0, 2))

    grid = (h, t // bq)
    out = pl.pallas_call(
        functools.partial(_attn_kernel, bq=bq, bk=bk, nbounds=nbounds),
        grid_spec=pltpu.PrefetchScalarGridSpec(
            num_scalar_prefetch=2,
            grid=grid,
            in_specs=[
                pl.BlockSpec((1, bq, d), lambda hh, ii, *_: (hh, ii, 0)),
                pl.BlockSpec((1, t, d), lambda hh, ii, *_: (hh // rep, 0, 0)),
                pl.BlockSpec((1, t, d), lambda hh, ii, *_: (hh // rep, 0, 0)),
            ],
            out_specs=pl.BlockSpec((1, bq, d), lambda hh, ii, *_: (hh, ii, 0)),
        ),
        out_shape=jax.ShapeDtypeStruct((h, t, d), jnp.float32),
    )(cu_seqlens_q.astype(jnp.int32), cu_seqlens_k.astype(jnp.int32), qh, kh, vh)
    return jnp.transpose(out, (1, 0, 2)).astype(q.dtype)
